# baseline (speedup 1.0000x reference)
import functools

import jax
import jax.numpy as jnp
from jax import lax
from jax.experimental import pallas as pl
from jax.experimental.pallas import tpu as pltpu
from jax.experimental.pallas import tpu_sc as plsc

N = 10000
NP = 10240
DEG = 16
ZERO = N
NC, NS = 2, 16
NW = NC * NS
C1 = NP // NW
B = 256
C2 = B // NW
NBUF = 4

_SC_PARAMS = dict(
    compiler_params=pltpu.CompilerParams(
        needs_layout_passes=False, use_tc_tiling_on_sc=False))


def _mesh():
    return plsc.VectorSubcoreMesh(
        core_axis_name="c", subcore_axis_name="s", num_cores=NC, num_subcores=NS)


def _wid():
    return lax.axis_index("s") * NC + lax.axis_index("c")


def _prep_body(feat_ref, w_ref, a1_ref, a2_ref, mlp_ref, w1_ref, w2_ref):
    mlp_ref[...] = jnp.dot(feat_ref[...], w_ref[...],
                           preferred_element_type=jnp.float32)
    for a_ref, o_ref in ((a1_ref, w1_ref), (a2_ref, w2_ref)):
        a = a_ref[...]
        e = jnp.exp(a - jnp.max(a, axis=0, keepdims=True))
        o_ref[...] = e / jnp.sum(e, axis=0, keepdims=True)


def _prep(featp, wp, a1t, a2t):
    return pl.pallas_call(
        _prep_body,
        out_shape=[
            jax.ShapeDtypeStruct((NP, 64), jnp.float32),
            jax.ShapeDtypeStruct((3, 128), jnp.float32),
            jax.ShapeDtypeStruct((3, 256), jnp.float32),
        ],
    )(featp, wp, a1t, a2t)


def _dedup_fire(adjv, n, k, idxv, gv, sem, invv, tmpv, table_hbm):
    iota = lax.iota(jnp.int32, 16)
    pidx = jnp.maximum(iota - 1, 0)
    first = iota == 0
    for r in range(3):
        a = adjv[n, pl.ds(r * 16, 16)]
        s, _ = plsc.sort_key_val(a, a)
        tmpv[...] = s
        prev = plsc.load_gather(tmpv, [pidx])
        m = (s != prev) | first
        cnt = plsc.all_reduce_population_count(m)
        invv[k, pl.ds(r * 16, 16)] = 1.0 / cnt.astype(jnp.float32)
        idxv[pl.ds(r * 16, 16)] = jnp.where(m, s, ZERO)
    pltpu.async_copy(table_hbm.at[idxv], gv, sem)


def _row_sums(gv, r, ngrp):
    acc = []
    for c in range(ngrp):
        a0 = gv[r * 16, pl.ds(c * 16, 16)]
        for j in range(1, 16):
            a0 = a0 + gv[r * 16 + j, pl.ds(c * 16, 16)]
        acc.append(a0)
    return acc


def _layer1_body(adj_hbm, table_hbm, w1_hbm, out_hbm,
                 adjv, selfv, outv, w1v, invv, tmpv,
                 idx0, idx1, idx2, idx3, g0, g1, g2, g3,
                 sem0, sem1, sem2, sem3):
    bufs = ((idx0, g0, sem0), (idx1, g1, sem1), (idx2, g2, sem2),
            (idx3, g3, sem3))
    base = _wid() * C1
    pltpu.sync_copy(adj_hbm.at[pl.ds(base, C1)], adjv.at[pl.ds(0, C1)])
    pltpu.sync_copy(table_hbm.at[pl.ds(base, C1)], selfv)
    pltpu.sync_copy(w1_hbm, w1v)
    zvec = jnp.full((16,), ZERO, jnp.int32)
    for rr in range(C1, C1 + NBUF):
        for c in range(3):
            adjv[rr, pl.ds(c * 16, 16)] = zvec

    def fire(n, k):
        idxv, gv, sem = bufs[k]
        _dedup_fire(adjv, n, k, idxv, gv, sem, invv, tmpv, table_hbm)

    def consume(n, k):
        gv = bufs[k][1]
        selfc = [selfv[n, pl.ds(c * 16, 16)] for c in range(4)]
        lo = [jnp.zeros((16,), jnp.float32)] * 4
        hi = [jnp.zeros((16,), jnp.float32)] * 4
        for r in range(3):
            sums = _row_sums(gv, r, 4)
            inv = invv[k, pl.ds(r * 16, 16)]
            for c in range(4):
                mean = sums[c] * inv
                lo[c] = lo[c] + mean * w1v[r, pl.ds(c * 16, 16)]
                hi[c] = hi[c] + (selfc[c] - mean) * w1v[r, pl.ds(64 + c * 16, 16)]
        for c in range(4):
            outv[n, pl.ds(c * 16, 16)] = lo[c]
            outv[n, pl.ds(64 + c * 16, 16)] = hi[c]

    for k in range(NBUF):
        fire(jnp.int32(k), k)

    def body(ig, carry):
        n0 = ig * jnp.int32(NBUF)
        for k in range(NBUF):
            idxv, gv, sem = bufs[k]
            pltpu.make_async_copy(table_hbm.at[idxv], gv, sem).wait()
            consume(n0 + k, k)
            fire(n0 + NBUF + k, k)
        return carry

    lax.fori_loop(jnp.int32(0), jnp.int32(C1 // NBUF), body, jnp.int32(0))
    for k in range(NBUF):
        idxv, gv, sem = bufs[k]
        pltpu.make_async_copy(table_hbm.at[idxv], gv, sem).wait()
    pltpu.sync_copy(outv, out_hbm.at[pl.ds(base, C1)])


def _layer1(adjcat, table1, w1t):
    f = functools.partial(
        pl.kernel,
        out_type=jax.ShapeDtypeStruct((NP, 128), jnp.float32),
        mesh=_mesh(),
        scratch_types=[
            pltpu.VMEM((C1 + NBUF, 48), jnp.int32),
            pltpu.VMEM((C1, 64), jnp.float32),
            pltpu.VMEM((C1, 128), jnp.float32),
            pltpu.VMEM((3, 128), jnp.float32),
            pltpu.VMEM((NBUF, 48), jnp.float32),
            pltpu.VMEM((16,), jnp.int32),
        ] + [pltpu.VMEM((48,), jnp.int32)] * NBUF
          + [pltpu.VMEM((48, 64), jnp.float32)] * NBUF
          + [pltpu.SemaphoreType.DMA] * NBUF,
        **_SC_PARAMS,
    )(_layer1_body)
    return f(adjcat, table1, w1t)


def _layer2_body(nodes_hbm, adj_hbm, t1_hbm, t2_hbm, w2_hbm, out_hbm,
                 nodesv, adjv, selfa, selfb, outv, w2v, invv, tmpv,
                 idx0, idx1, idx2, idx3, g0, g1, g2, g3,
                 sem0, sem1, sem2, sem3):
    bufs = ((idx0, g0, sem0), (idx1, g1, sem1), (idx2, g2, sem2),
            (idx3, g3, sem3))
    base = _wid() * C2
    pltpu.sync_copy(nodes_hbm.at[pl.ds(base, C2)], nodesv)
    pltpu.sync_copy(w2_hbm, w2v)
    pltpu.async_copy(adj_hbm.at[nodesv], adjv.at[pl.ds(0, C2)], sem0)
    pltpu.async_copy(t1_hbm.at[nodesv], selfa, sem1)
    pltpu.async_copy(t2_hbm.at[nodesv], selfb, sem2)
    pltpu.make_async_copy(adj_hbm.at[nodesv], adjv.at[pl.ds(0, C2)], sem0).wait()
    pltpu.make_async_copy(t1_hbm.at[nodesv], selfa, sem1).wait()
    pltpu.make_async_copy(t2_hbm.at[nodesv], selfb, sem2).wait()
    zvec = jnp.full((16,), ZERO, jnp.int32)
    for rr in range(C2, C2 + NBUF):
        for c in range(3):
            adjv[rr, pl.ds(c * 16, 16)] = zvec

    def fire(n, k):
        idxv, gv, sem = bufs[k]
        _dedup_fire(adjv, n, k, idxv, gv, sem, invv, tmpv, t2_hbm)

    def consume(n, k):
        gv = bufs[k][1]
        selfc = [selfb[n, pl.ds(c * 16, 16)] for c in range(8)]
        lo = [jnp.zeros((16,), jnp.float32)] * 8
        hi = [jnp.zeros((16,), jnp.float32)] * 8
        for r in range(3):
            sums = _row_sums(gv, r, 8)
            inv = invv[k, pl.ds(r * 16, 16)]
            for c in range(8):
                mean = sums[c] * inv
                lo[c] = lo[c] + mean * w2v[r, pl.ds(c * 16, 16)]
                hi[c] = hi[c] + (selfc[c] - mean) * w2v[r, pl.ds(128 + c * 16, 16)]
        for c in range(4):
            outv[n, pl.ds(c * 16, 16)] = selfa[n, pl.ds(c * 16, 16)]
        for c in range(8):
            outv[n, pl.ds(64 + c * 16, 16)] = selfc[c]
            outv[n, pl.ds(192 + c * 16, 16)] = lo[c]
            outv[n, pl.ds(320 + c * 16, 16)] = hi[c]

    for k in range(NBUF):
        fire(jnp.int32(k), k)

    def body(ig, carry):
        n0 = ig * jnp.int32(NBUF)
        for k in range(NBUF):
            idxv, gv, sem = bufs[k]
            pltpu.make_async_copy(t2_hbm.at[idxv], gv, sem).wait()
            consume(n0 + k, k)
            fire(n0 + NBUF + k, k)
        return carry

    lax.fori_loop(jnp.int32(0), jnp.int32(C2 // NBUF), body, jnp.int32(0))
    for k in range(NBUF):
        idxv, gv, sem = bufs[k]
        pltpu.make_async_copy(t2_hbm.at[idxv], gv, sem).wait()
    pltpu.sync_copy(outv, out_hbm.at[pl.ds(base, C2)])


def _layer2(nodes32, adjcat, table1, table2, w2t):
    f = functools.partial(
        pl.kernel,
        out_type=jax.ShapeDtypeStruct((B, 448), jnp.float32),
        mesh=_mesh(),
        scratch_types=[
            pltpu.VMEM((C2,), jnp.int32),
            pltpu.VMEM((C2 + NBUF, 48), jnp.int32),
            pltpu.VMEM((C2, 64), jnp.float32),
            pltpu.VMEM((C2, 128), jnp.float32),
            pltpu.VMEM((C2, 448), jnp.float32),
            pltpu.VMEM((3, 256), jnp.float32),
            pltpu.VMEM((NBUF, 48), jnp.float32),
            pltpu.VMEM((16,), jnp.int32),
        ] + [pltpu.VMEM((48,), jnp.int32)] * NBUF
          + [pltpu.VMEM((48, 128), jnp.float32)] * NBUF
          + [pltpu.SemaphoreType.DMA] * NBUF,
        **_SC_PARAMS,
    )(_layer2_body)
    return f(nodes32, adjcat, table1, table2, w2t)


def _final_body(x_ref, w_ref, p_ref, o_ref):
    o_ref[...] = jnp.dot(x_ref[...], w_ref[...],
                         preferred_element_type=jnp.float32) + jnp.log(p_ref[...])


def _final(out2, linear_W, prior):
    return pl.pallas_call(
        _final_body,
        out_shape=jax.ShapeDtypeStruct((B, 2), jnp.float32),
    )(out2, linear_W, prior.reshape(1, 2))


@jax.jit
def kernel(nodes, adj_lists, feat_data, prior, mlp_W, alpha1, alpha2, linear_W):
    nodes32 = nodes.astype(jnp.int32)
    adj32 = adj_lists.astype(jnp.int32)
    adjcat = jnp.transpose(adj32, (1, 0, 2)).reshape(N, 3 * DEG)
    adjcat = jnp.pad(adjcat, ((0, NP - N), (0, 0)), constant_values=ZERO)
    featp = jnp.pad(feat_data, ((0, NP - N), (0, 32 - 25)))
    wp = jnp.pad(mlp_W, ((0, 32 - 25), (0, 0)))
    table1, w1t, w2t = _prep(featp, wp, alpha1.T, alpha2.T)
    table2 = _layer1(adjcat, table1, w1t)
    out2 = _layer2(nodes32, adjcat, table1, table2, w2t)
    y = _final(out2, linear_W, prior)
    return y.astype(jnp.float64)


# speedup vs baseline: 1.2949x; 1.2949x over previous
import functools

import jax
import jax.numpy as jnp
from jax import lax
from jax.experimental import pallas as pl
from jax.experimental.pallas import tpu as pltpu
from jax.experimental.pallas import tpu_sc as plsc

N = 10000
NP = 10240
DEG = 16
ZERO = N
NC, NS = 2, 16
NW = NC * NS
C1 = NP // NW
B = 256
C2 = B // NW
NBUF = 2

_SC_PARAMS = dict(
    compiler_params=pltpu.CompilerParams(
        needs_layout_passes=False, use_tc_tiling_on_sc=False))


def _mesh():
    return plsc.VectorSubcoreMesh(
        core_axis_name="c", subcore_axis_name="s", num_cores=NC, num_subcores=NS)


def _wid():
    return lax.axis_index("s") * NC + lax.axis_index("c")


def _prep_body(feat_ref, w_ref, a1_ref, a2_ref, mlp_ref, w1_ref, w2_ref):
    mlp_ref[...] = jnp.dot(feat_ref[...], w_ref[...],
                           preferred_element_type=jnp.float32)
    for a_ref, o_ref in ((a1_ref, w1_ref), (a2_ref, w2_ref)):
        a = a_ref[...]
        e = jnp.exp(a - jnp.max(a, axis=0, keepdims=True))
        o_ref[...] = e / jnp.sum(e, axis=0, keepdims=True)


def _prep(featp, wp, a1t, a2t):
    return pl.pallas_call(
        _prep_body,
        out_shape=[
            jax.ShapeDtypeStruct((NP, 64), jnp.float32),
            jax.ShapeDtypeStruct((3, 128), jnp.float32),
            jax.ShapeDtypeStruct((3, 256), jnp.float32),
        ],
    )(featp, wp, a1t, a2t)


def _dedup_fire(adjv, n, k, idxv, gv, sem, invv, tmpv, table_hbm):
    iota = lax.iota(jnp.int32, 16)
    pidx = jnp.maximum(iota - 1, 0)
    first = iota == 0
    for r in range(3):
        a = adjv[n, pl.ds(r * 16, 16)]
        s, _ = plsc.sort_key_val(a, a)
        tmpv[...] = s
        prev = plsc.load_gather(tmpv, [pidx])
        m = (s != prev) | first
        cnt = plsc.all_reduce_population_count(m)
        invv[k, pl.ds(r * 16, 16)] = 1.0 / cnt.astype(jnp.float32)
        idxv[pl.ds(r * 16, 16)] = jnp.where(m, s, ZERO)
    pltpu.async_copy(table_hbm.at[idxv], gv, sem)


def _row_sums(gv, r, ngrp):
    acc = []
    for c in range(ngrp):
        a0 = gv[r * 16, pl.ds(c * 16, 16)]
        for j in range(1, 16):
            a0 = a0 + gv[r * 16 + j, pl.ds(c * 16, 16)]
        acc.append(a0)
    return acc


def _layer1_body(adj_hbm, table_hbm, w1_hbm, out_hbm,
                 adjv, selfv, outv, w1v, invv, tmpv, *rest):
    bufs = tuple((rest[k], rest[NBUF + k], rest[2 * NBUF + k])
                 for k in range(NBUF))
    base = _wid() * C1
    pltpu.sync_copy(adj_hbm.at[pl.ds(base, C1)], adjv.at[pl.ds(0, C1)])
    pltpu.sync_copy(table_hbm.at[pl.ds(base, C1)], selfv)
    pltpu.sync_copy(w1_hbm, w1v)
    zvec = jnp.full((16,), ZERO, jnp.int32)
    for rr in range(C1, C1 + NBUF):
        for c in range(3):
            adjv[rr, pl.ds(c * 16, 16)] = zvec

    def fire(n, k):
        idxv, gv, sem = bufs[k]
        _dedup_fire(adjv, n, k, idxv, gv, sem, invv, tmpv, table_hbm)

    def consume(n, k):
        gv = bufs[k][1]
        selfc = [selfv[n, pl.ds(c * 16, 16)] for c in range(4)]
        lo = [jnp.zeros((16,), jnp.float32)] * 4
        hi = [jnp.zeros((16,), jnp.float32)] * 4
        for r in range(3):
            sums = _row_sums(gv, r, 4)
            inv = invv[k, pl.ds(r * 16, 16)]
            for c in range(4):
                mean = sums[c] * inv
                lo[c] = lo[c] + mean * w1v[r, pl.ds(c * 16, 16)]
                hi[c] = hi[c] + (selfc[c] - mean) * w1v[r, pl.ds(64 + c * 16, 16)]
        for c in range(4):
            outv[n, pl.ds(c * 16, 16)] = lo[c]
            outv[n, pl.ds(64 + c * 16, 16)] = hi[c]

    for k in range(NBUF):
        fire(jnp.int32(k), k)

    def body(ig, carry):
        n0 = ig * jnp.int32(NBUF)
        for k in range(NBUF):
            idxv, gv, sem = bufs[k]
            pltpu.make_async_copy(table_hbm.at[idxv], gv, sem).wait()
            consume(n0 + k, k)
            fire(n0 + NBUF + k, k)
        return carry

    lax.fori_loop(jnp.int32(0), jnp.int32(C1 // NBUF), body, jnp.int32(0))
    for k in range(NBUF):
        idxv, gv, sem = bufs[k]
        pltpu.make_async_copy(table_hbm.at[idxv], gv, sem).wait()
    pltpu.sync_copy(outv, out_hbm.at[pl.ds(base, C1)])


def _layer1(adjcat, table1, w1t):
    f = functools.partial(
        pl.kernel,
        out_type=jax.ShapeDtypeStruct((NP, 128), jnp.float32),
        mesh=_mesh(),
        scratch_types=[
            pltpu.VMEM((C1 + NBUF, 48), jnp.int32),
            pltpu.VMEM((C1, 64), jnp.float32),
            pltpu.VMEM((C1, 128), jnp.float32),
            pltpu.VMEM((3, 128), jnp.float32),
            pltpu.VMEM((NBUF, 48), jnp.float32),
            pltpu.VMEM((16,), jnp.int32),
        ] + [pltpu.VMEM((48,), jnp.int32)] * NBUF
          + [pltpu.VMEM((48, 64), jnp.float32)] * NBUF
          + [pltpu.SemaphoreType.DMA] * NBUF,
        **_SC_PARAMS,
    )(_layer1_body)
    return f(adjcat, table1, w1t)


def _layer2_body(nodes_hbm, adj_hbm, t1_hbm, t2_hbm, w2_hbm, out_hbm,
                 nodesv, adjv, selfa, selfb, outv, w2v, invv, tmpv, *rest):
    bufs = tuple((rest[k], rest[NBUF + k], rest[2 * NBUF + k])
                 for k in range(NBUF))
    base = _wid() * C2
    pltpu.sync_copy(nodes_hbm.at[pl.ds(base, C2)], nodesv)
    pltpu.sync_copy(w2_hbm, w2v)
    sA, sB = bufs[0][2], bufs[1][2]
    pltpu.async_copy(adj_hbm.at[nodesv], adjv.at[pl.ds(0, C2)], sA)
    pltpu.async_copy(t1_hbm.at[nodesv], selfa, sB)
    pltpu.make_async_copy(adj_hbm.at[nodesv], adjv.at[pl.ds(0, C2)], sA).wait()
    pltpu.async_copy(t2_hbm.at[nodesv], selfb, sA)
    pltpu.make_async_copy(t1_hbm.at[nodesv], selfa, sB).wait()
    pltpu.make_async_copy(t2_hbm.at[nodesv], selfb, sA).wait()
    zvec = jnp.full((16,), ZERO, jnp.int32)
    for rr in range(C2, C2 + NBUF):
        for c in range(3):
            adjv[rr, pl.ds(c * 16, 16)] = zvec

    def fire(n, k):
        idxv, gv, sem = bufs[k]
        _dedup_fire(adjv, n, k, idxv, gv, sem, invv, tmpv, t2_hbm)

    def consume(n, k):
        gv = bufs[k][1]
        selfc = [selfb[n, pl.ds(c * 16, 16)] for c in range(8)]
        lo = [jnp.zeros((16,), jnp.float32)] * 8
        hi = [jnp.zeros((16,), jnp.float32)] * 8
        for r in range(3):
            sums = _row_sums(gv, r, 8)
            inv = invv[k, pl.ds(r * 16, 16)]
            for c in range(8):
                mean = sums[c] * inv
                lo[c] = lo[c] + mean * w2v[r, pl.ds(c * 16, 16)]
                hi[c] = hi[c] + (selfc[c] - mean) * w2v[r, pl.ds(128 + c * 16, 16)]
        for c in range(4):
            outv[n, pl.ds(c * 16, 16)] = selfa[n, pl.ds(c * 16, 16)]
        for c in range(8):
            outv[n, pl.ds(64 + c * 16, 16)] = selfc[c]
            outv[n, pl.ds(192 + c * 16, 16)] = lo[c]
            outv[n, pl.ds(320 + c * 16, 16)] = hi[c]

    for k in range(NBUF):
        fire(jnp.int32(k), k)

    def body(ig, carry):
        n0 = ig * jnp.int32(NBUF)
        for k in range(NBUF):
            idxv, gv, sem = bufs[k]
            pltpu.make_async_copy(t2_hbm.at[idxv], gv, sem).wait()
            consume(n0 + k, k)
            fire(n0 + NBUF + k, k)
        return carry

    lax.fori_loop(jnp.int32(0), jnp.int32(C2 // NBUF), body, jnp.int32(0))
    for k in range(NBUF):
        idxv, gv, sem = bufs[k]
        pltpu.make_async_copy(t2_hbm.at[idxv], gv, sem).wait()
    pltpu.sync_copy(outv, out_hbm.at[pl.ds(base, C2)])


def _layer2(nodes32, adjcat, table1, table2, w2t):
    f = functools.partial(
        pl.kernel,
        out_type=jax.ShapeDtypeStruct((B, 448), jnp.float32),
        mesh=_mesh(),
        scratch_types=[
            pltpu.VMEM((C2,), jnp.int32),
            pltpu.VMEM((C2 + NBUF, 48), jnp.int32),
            pltpu.VMEM((C2, 64), jnp.float32),
            pltpu.VMEM((C2, 128), jnp.float32),
            pltpu.VMEM((C2, 448), jnp.float32),
            pltpu.VMEM((3, 256), jnp.float32),
            pltpu.VMEM((NBUF, 48), jnp.float32),
            pltpu.VMEM((16,), jnp.int32),
        ] + [pltpu.VMEM((48,), jnp.int32)] * NBUF
          + [pltpu.VMEM((48, 128), jnp.float32)] * NBUF
          + [pltpu.SemaphoreType.DMA] * NBUF,
        **_SC_PARAMS,
    )(_layer2_body)
    return f(nodes32, adjcat, table1, table2, w2t)


def _final_body(x_ref, w_ref, p_ref, o_ref):
    o_ref[...] = jnp.dot(x_ref[...], w_ref[...],
                         preferred_element_type=jnp.float32) + jnp.log(p_ref[...])


def _final(out2, linear_W, prior):
    return pl.pallas_call(
        _final_body,
        out_shape=jax.ShapeDtypeStruct((B, 2), jnp.float32),
    )(out2, linear_W, prior.reshape(1, 2))


@jax.jit
def kernel(nodes, adj_lists, feat_data, prior, mlp_W, alpha1, alpha2, linear_W):
    nodes32 = nodes.astype(jnp.int32)
    adj32 = adj_lists.astype(jnp.int32)
    adjcat = jnp.transpose(adj32, (1, 0, 2)).reshape(N, 3 * DEG)
    adjcat = jnp.pad(adjcat, ((0, NP - N), (0, 0)), constant_values=ZERO)
    featp = jnp.pad(feat_data, ((0, NP - N), (0, 32 - 25)))
    wp = jnp.pad(mlp_W, ((0, 32 - 25), (0, 0)))
    table1, w1t, w2t = _prep(featp, wp, alpha1.T, alpha2.T)
    table2 = _layer1(adjcat, table1, w1t)
    out2 = _layer2(nodes32, adjcat, table1, table2, w2t)
    y = _final(out2, linear_W, prior)
    return y.astype(jnp.float64)


# speedup vs baseline: 4.8747x; 3.7646x over previous
import functools

import jax
import jax.numpy as jnp
from jax import lax
from jax.experimental import pallas as pl
from jax.experimental.pallas import tpu as pltpu
from jax.experimental.pallas import tpu_sc as plsc

N = 10000
NP = 10240
DEG = 16
ZERO = N
NC, NS = 2, 16
NW = NC * NS
C1 = NP // NW
SEG = NP // NS
B = 256
C2 = B // NW

_SC_PARAMS = dict(
    compiler_params=pltpu.CompilerParams(
        needs_layout_passes=False, use_tc_tiling_on_sc=False))


def _mesh():
    return plsc.VectorSubcoreMesh(
        core_axis_name="c", subcore_axis_name="s", num_cores=NC, num_subcores=NS)


def _wid():
    return lax.axis_index("s") * NC + lax.axis_index("c")


def _prep_body(feat_ref, w_ref, a1_ref, a2_ref, mlp_ref, w1_ref, w2_ref):
    mlp_ref[...] = jnp.dot(feat_ref[...], w_ref[...],
                           preferred_element_type=jnp.float32)
    for a_ref, o_ref in ((a1_ref, w1_ref), (a2_ref, w2_ref)):
        a = a_ref[...]
        e = jnp.exp(a - jnp.max(a, axis=0, keepdims=True))
        o_ref[...] = e / jnp.sum(e, axis=0, keepdims=True)


def _prep(featp, wp, a1t, a2t):
    return pl.pallas_call(
        _prep_body,
        out_shape=[
            jax.ShapeDtypeStruct((NP, 64), jnp.float32),
            jax.ShapeDtypeStruct((3, 128), jnp.float32),
            jax.ShapeDtypeStruct((3, 256), jnp.float32),
        ],
    )(featp, wp, a1t, a2t)


def _stage_shared(table_hbm, sh_table):
    seg = lax.axis_index("s") * jnp.int32(SEG)
    pltpu.sync_copy(table_hbm.at[pl.ds(seg, SEG)], sh_table.at[pl.ds(seg, SEG)])
    plsc.subcore_barrier()


def _dedup_fire(adjv, n, idxv, gv, sem, src_table):
    iota = lax.iota(jnp.int32, 16)
    pidx = jnp.maximum(iota - 1, 0)
    first = iota == 0
    invs = []
    for r in range(3):
        a = adjv[n, pl.ds(r * 16, 16)]
        s, _ = plsc.sort_key_val(a, a)
        idxv[pl.ds(r * 16, 16)] = s
        prev = plsc.load_gather(idxv, [jnp.full((16,), r * 16, jnp.int32) + pidx])
        m = (s != prev) | first
        cnt = plsc.all_reduce_population_count(m)
        invs.append(1.0 / cnt.astype(jnp.float32))
        idxv[pl.ds(r * 16, 16)] = jnp.where(m, s, ZERO)
    pltpu.async_copy(src_table.at[idxv], gv, sem)
    return tuple(invs)


def _row_sums(gv, r, ngrp):
    acc = []
    for c in range(ngrp):
        a0 = gv[r * 16, pl.ds(c * 16, 16)]
        for j in range(1, 16):
            a0 = a0 + gv[r * 16 + j, pl.ds(c * 16, 16)]
        acc.append(a0)
    return acc


def _layer1_body(adj_hbm, table_hbm, w1_hbm, out_hbm,
                 sh_table, adjv, selfv, outv, w1v,
                 idx0, idx1, g0, g1, sem0, sem1):
    base = _wid() * C1
    pltpu.sync_copy(adj_hbm.at[pl.ds(base, C1)], adjv.at[pl.ds(0, C1)])
    pltpu.sync_copy(table_hbm.at[pl.ds(base, C1)], selfv)
    pltpu.sync_copy(w1_hbm, w1v)
    zvec = jnp.full((16,), ZERO, jnp.int32)
    for rr in range(C1, C1 + 2):
        for c in range(3):
            adjv[rr, pl.ds(c * 16, 16)] = zvec
    _stage_shared(table_hbm, sh_table)

    def consume(n, gv, invs):
        selfc = [selfv[n, pl.ds(c * 16, 16)] for c in range(4)]
        lo = [jnp.zeros((16,), jnp.float32)] * 4
        hi = [jnp.zeros((16,), jnp.float32)] * 4
        for r in range(3):
            sums = _row_sums(gv, r, 4)
            for c in range(4):
                mean = sums[c] * invs[r]
                lo[c] = lo[c] + mean * w1v[r, pl.ds(c * 16, 16)]
                hi[c] = hi[c] + (selfc[c] - mean) * w1v[r, pl.ds(64 + c * 16, 16)]
        for c in range(4):
            outv[n, pl.ds(c * 16, 16)] = lo[c]
            outv[n, pl.ds(64 + c * 16, 16)] = hi[c]

    inv_a = _dedup_fire(adjv, jnp.int32(0), idx0, g0, sem0, sh_table)

    def body(i2, carry):
        n = i2 * jnp.int32(2)
        inv_b = _dedup_fire(adjv, n + 1, idx1, g1, sem1, sh_table)
        pltpu.make_async_copy(sh_table.at[idx0], g0, sem0).wait()
        consume(n, g0, carry)
        inv_c = _dedup_fire(adjv, n + 2, idx0, g0, sem0, sh_table)
        pltpu.make_async_copy(sh_table.at[idx1], g1, sem1).wait()
        consume(n + 1, g1, inv_b)
        return inv_c

    lax.fori_loop(jnp.int32(0), jnp.int32(C1 // 2), body, inv_a)
    pltpu.make_async_copy(sh_table.at[idx0], g0, sem0).wait()
    pltpu.sync_copy(outv, out_hbm.at[pl.ds(base, C1)])


def _layer1(adjcat, table1, w1t):
    f = functools.partial(
        pl.kernel,
        out_type=jax.ShapeDtypeStruct((NP, 128), jnp.float32),
        mesh=_mesh(),
        scratch_types=[
            pltpu.VMEM_SHARED((NP, 64), jnp.float32),
            pltpu.VMEM((C1 + 2, 48), jnp.int32),
            pltpu.VMEM((C1, 64), jnp.float32),
            pltpu.VMEM((C1, 128), jnp.float32),
            pltpu.VMEM((3, 128), jnp.float32),
            pltpu.VMEM((48,), jnp.int32),
            pltpu.VMEM((48,), jnp.int32),
            pltpu.VMEM((48, 64), jnp.float32),
            pltpu.VMEM((48, 64), jnp.float32),
            pltpu.SemaphoreType.DMA,
            pltpu.SemaphoreType.DMA,
        ],
        **_SC_PARAMS,
    )(_layer1_body)
    return f(adjcat, table1, w1t)


def _layer2_body(nodes_hbm, adj_hbm, t1_hbm, t2_hbm, w2_hbm, out_hbm,
                 sh_table, nodesv, adjv, selfa, selfb, outv, w2v,
                 idx0, idx1, g0, g1, sem0, sem1):
    base = _wid() * C2
    pltpu.sync_copy(nodes_hbm.at[pl.ds(base, C2)], nodesv)
    pltpu.sync_copy(w2_hbm, w2v)
    pltpu.async_copy(adj_hbm.at[nodesv], adjv.at[pl.ds(0, C2)], sem0)
    pltpu.async_copy(t1_hbm.at[nodesv], selfa, sem1)
    pltpu.make_async_copy(adj_hbm.at[nodesv], adjv.at[pl.ds(0, C2)], sem0).wait()
    pltpu.async_copy(t2_hbm.at[nodesv], selfb, sem0)
    pltpu.make_async_copy(t1_hbm.at[nodesv], selfa, sem1).wait()
    pltpu.make_async_copy(t2_hbm.at[nodesv], selfb, sem0).wait()
    zvec = jnp.full((16,), ZERO, jnp.int32)
    for rr in range(C2, C2 + 2):
        for c in range(3):
            adjv[rr, pl.ds(c * 16, 16)] = zvec
    _stage_shared(t2_hbm, sh_table)

    def consume(n, gv, invs):
        selfc = [selfb[n, pl.ds(c * 16, 16)] for c in range(8)]
        lo = [jnp.zeros((16,), jnp.float32)] * 8
        hi = [jnp.zeros((16,), jnp.float32)] * 8
        for r in range(3):
            sums = _row_sums(gv, r, 8)
            for c in range(8):
                mean = sums[c] * invs[r]
                lo[c] = lo[c] + mean * w2v[r, pl.ds(c * 16, 16)]
                hi[c] = hi[c] + (selfc[c] - mean) * w2v[r, pl.ds(128 + c * 16, 16)]
        for c in range(4):
            outv[n, pl.ds(c * 16, 16)] = selfa[n, pl.ds(c * 16, 16)]
        for c in range(8):
            outv[n, pl.ds(64 + c * 16, 16)] = selfc[c]
            outv[n, pl.ds(192 + c * 16, 16)] = lo[c]
            outv[n, pl.ds(320 + c * 16, 16)] = hi[c]

    inv_a = _dedup_fire(adjv, jnp.int32(0), idx0, g0, sem0, sh_table)

    def body(i2, carry):
        n = i2 * jnp.int32(2)
        inv_b = _dedup_fire(adjv, n + 1, idx1, g1, sem1, sh_table)
        pltpu.make_async_copy(sh_table.at[idx0], g0, sem0).wait()
        consume(n, g0, carry)
        inv_c = _dedup_fire(adjv, n + 2, idx0, g0, sem0, sh_table)
        pltpu.make_async_copy(sh_table.at[idx1], g1, sem1).wait()
        consume(n + 1, g1, inv_b)
        return inv_c

    lax.fori_loop(jnp.int32(0), jnp.int32(C2 // 2), body, inv_a)
    pltpu.make_async_copy(sh_table.at[idx0], g0, sem0).wait()
    pltpu.sync_copy(outv, out_hbm.at[pl.ds(base, C2)])


def _layer2(nodes32, adjcat, table1, table2, w2t):
    f = functools.partial(
        pl.kernel,
        out_type=jax.ShapeDtypeStruct((B, 448), jnp.float32),
        mesh=_mesh(),
        scratch_types=[
            pltpu.VMEM_SHARED((NP, 128), jnp.float32),
            pltpu.VMEM((C2,), jnp.int32),
            pltpu.VMEM((C2 + 2, 48), jnp.int32),
            pltpu.VMEM((C2, 64), jnp.float32),
            pltpu.VMEM((C2, 128), jnp.float32),
            pltpu.VMEM((C2, 448), jnp.float32),
            pltpu.VMEM((3, 256), jnp.float32),
            pltpu.VMEM((48,), jnp.int32),
            pltpu.VMEM((48,), jnp.int32),
            pltpu.VMEM((48, 128), jnp.float32),
            pltpu.VMEM((48, 128), jnp.float32),
            pltpu.SemaphoreType.DMA,
            pltpu.SemaphoreType.DMA,
        ],
        **_SC_PARAMS,
    )(_layer2_body)
    return f(nodes32, adjcat, table1, table2, w2t)


def _final_body(x_ref, w_ref, p_ref, o_ref):
    o_ref[...] = jnp.dot(x_ref[...], w_ref[...],
                         preferred_element_type=jnp.float32) + jnp.log(p_ref[...])


def _final(out2, linear_W, prior):
    return pl.pallas_call(
        _final_body,
        out_shape=jax.ShapeDtypeStruct((B, 2), jnp.float32),
    )(out2, linear_W, prior.reshape(1, 2))


@jax.jit
def kernel(nodes, adj_lists, feat_data, prior, mlp_W, alpha1, alpha2, linear_W):
    nodes32 = nodes.astype(jnp.int32)
    adj32 = adj_lists.astype(jnp.int32)
    adjcat = jnp.transpose(adj32, (1, 0, 2)).reshape(N, 3 * DEG)
    adjcat = jnp.pad(adjcat, ((0, NP - N), (0, 0)), constant_values=ZERO)
    featp = jnp.pad(feat_data, ((0, NP - N), (0, 32 - 25)))
    wp = jnp.pad(mlp_W, ((0, 32 - 25), (0, 0)))
    table1, w1t, w2t = _prep(featp, wp, alpha1.T, alpha2.T)
    table2 = _layer1(adjcat, table1, w1t)
    out2 = _layer2(nodes32, adjcat, table1, table2, w2t)
    y = _final(out2, linear_W, prior)
    return y.astype(jnp.float64)


# speedup vs baseline: 5.0765x; 1.0414x over previous
import functools

import jax
import jax.numpy as jnp
from jax import lax
from jax.experimental import pallas as pl
from jax.experimental.pallas import tpu as pltpu
from jax.experimental.pallas import tpu_sc as plsc

N = 10000
NP = 10240
DEG = 16
ZERO = N
NC, NS = 2, 16
NW = NC * NS
C1 = NP // NW
SEG = NP // NS
B = 256
C2 = B // NW

_SC_PARAMS = dict(
    compiler_params=pltpu.CompilerParams(
        needs_layout_passes=False, use_tc_tiling_on_sc=False))


def _mesh():
    return plsc.VectorSubcoreMesh(
        core_axis_name="c", subcore_axis_name="s", num_cores=NC, num_subcores=NS)


def _wid():
    return lax.axis_index("s") * NC + lax.axis_index("c")


def _prep_body(feat_ref, w_ref, a1_ref, a2_ref, mlp_ref, w1_ref, w2_ref):
    mlp_ref[...] = jnp.dot(feat_ref[...], w_ref[...],
                           preferred_element_type=jnp.float32)
    for a_ref, o_ref in ((a1_ref, w1_ref), (a2_ref, w2_ref)):
        a = a_ref[...]
        e = jnp.exp(a - jnp.max(a, axis=0, keepdims=True))
        o_ref[...] = e / jnp.sum(e, axis=0, keepdims=True)


def _prep(featp, wp, a1t, a2t):
    return pl.pallas_call(
        _prep_body,
        out_shape=[
            jax.ShapeDtypeStruct((NP, 64), jnp.float32),
            jax.ShapeDtypeStruct((3, 128), jnp.float32),
            jax.ShapeDtypeStruct((3, 256), jnp.float32),
        ],
    )(featp, wp, a1t, a2t)


def _stage_shared(table_hbm, sh_table):
    seg = lax.axis_index("s") * jnp.int32(SEG)
    pltpu.sync_copy(table_hbm.at[pl.ds(seg, SEG)], sh_table.at[pl.ds(seg, SEG)])
    plsc.subcore_barrier()


def _dedup_fire(adjv, n, idxv, gv, sem, src_table):
    invs = []
    for r in range(3):
        a = adjv[n, pl.ds(r * 16, 16)]
        _, m = plsc.scan_count(a)
        cnt = plsc.all_reduce_population_count(m)
        invs.append(1.0 / cnt.astype(jnp.float32))
        idxv[pl.ds(r * 16, 16)] = jnp.where(m, a, ZERO)
    pltpu.async_copy(src_table.at[idxv], gv, sem)
    return tuple(invs)


def _row_sums(gv, r, ngrp):
    acc = []
    for c in range(ngrp):
        a0 = gv[r * 16, pl.ds(c * 16, 16)]
        for j in range(1, 16):
            a0 = a0 + gv[r * 16 + j, pl.ds(c * 16, 16)]
        acc.append(a0)
    return acc


def _layer1_body(adj_hbm, table_hbm, w1_hbm, out_hbm,
                 sh_table, adjv, selfv, outv, w1v,
                 idx0, idx1, g0, g1, sem0, sem1):
    base = _wid() * C1
    pltpu.sync_copy(adj_hbm.at[pl.ds(base, C1)], adjv.at[pl.ds(0, C1)])
    pltpu.sync_copy(table_hbm.at[pl.ds(base, C1)], selfv)
    pltpu.sync_copy(w1_hbm, w1v)
    zvec = jnp.full((16,), ZERO, jnp.int32)
    for rr in range(C1, C1 + 2):
        for c in range(3):
            adjv[rr, pl.ds(c * 16, 16)] = zvec
    _stage_shared(table_hbm, sh_table)

    wlo = [[w1v[r, pl.ds(c * 16, 16)] for c in range(4)] for r in range(3)]
    whi = [[w1v[r, pl.ds(64 + c * 16, 16)] for c in range(4)] for r in range(3)]

    def consume(n, gv, invs):
        selfc = [selfv[n, pl.ds(c * 16, 16)] for c in range(4)]
        lo = [jnp.zeros((16,), jnp.float32)] * 4
        hi = [jnp.zeros((16,), jnp.float32)] * 4
        for r in range(3):
            sums = _row_sums(gv, r, 4)
            for c in range(4):
                mean = sums[c] * invs[r]
                lo[c] = lo[c] + mean * wlo[r][c]
                hi[c] = hi[c] + (selfc[c] - mean) * whi[r][c]
        for c in range(4):
            outv[n, pl.ds(c * 16, 16)] = lo[c]
            outv[n, pl.ds(64 + c * 16, 16)] = hi[c]

    inv_a = _dedup_fire(adjv, jnp.int32(0), idx0, g0, sem0, sh_table)

    def body(i2, carry):
        n = i2 * jnp.int32(2)
        inv_b = _dedup_fire(adjv, n + 1, idx1, g1, sem1, sh_table)
        pltpu.make_async_copy(sh_table.at[idx0], g0, sem0).wait()
        consume(n, g0, carry)
        inv_c = _dedup_fire(adjv, n + 2, idx0, g0, sem0, sh_table)
        pltpu.make_async_copy(sh_table.at[idx1], g1, sem1).wait()
        consume(n + 1, g1, inv_b)
        return inv_c

    lax.fori_loop(jnp.int32(0), jnp.int32(C1 // 2), body, inv_a)
    pltpu.make_async_copy(sh_table.at[idx0], g0, sem0).wait()
    pltpu.sync_copy(outv, out_hbm.at[pl.ds(base, C1)])


def _layer1(adjcat, table1, w1t):
    f = functools.partial(
        pl.kernel,
        out_type=jax.ShapeDtypeStruct((NP, 128), jnp.float32),
        mesh=_mesh(),
        scratch_types=[
            pltpu.VMEM_SHARED((NP, 64), jnp.float32),
            pltpu.VMEM((C1 + 2, 48), jnp.int32),
            pltpu.VMEM((C1, 64), jnp.float32),
            pltpu.VMEM((C1, 128), jnp.float32),
            pltpu.VMEM((3, 128), jnp.float32),
            pltpu.VMEM((48,), jnp.int32),
            pltpu.VMEM((48,), jnp.int32),
            pltpu.VMEM((48, 64), jnp.float32),
            pltpu.VMEM((48, 64), jnp.float32),
            pltpu.SemaphoreType.DMA,
            pltpu.SemaphoreType.DMA,
        ],
        **_SC_PARAMS,
    )(_layer1_body)
    return f(adjcat, table1, w1t)


def _layer2_body(nodes_hbm, adj_hbm, t1_hbm, t2_hbm, w2_hbm, out_hbm,
                 sh_table, nodesv, adjv, selfa, selfb, outv, w2v,
                 idx0, idx1, g0, g1, sem0, sem1):
    base = _wid() * C2
    pltpu.sync_copy(nodes_hbm.at[pl.ds(base, C2)], nodesv)
    pltpu.sync_copy(w2_hbm, w2v)
    pltpu.async_copy(adj_hbm.at[nodesv], adjv.at[pl.ds(0, C2)], sem0)
    pltpu.async_copy(t1_hbm.at[nodesv], selfa, sem1)
    pltpu.async_copy(t2_hbm.at[nodesv], selfb, sem1)
    _stage_shared(t2_hbm, sh_table)
    pltpu.make_async_copy(adj_hbm.at[nodesv], adjv.at[pl.ds(0, C2)], sem0).wait()
    pltpu.make_async_copy(t1_hbm.at[nodesv], selfa, sem1).wait()
    pltpu.make_async_copy(t2_hbm.at[nodesv], selfb, sem1).wait()
    zvec = jnp.full((16,), ZERO, jnp.int32)
    for rr in range(C2, C2 + 2):
        for c in range(3):
            adjv[rr, pl.ds(c * 16, 16)] = zvec

    def consume(n, gv, invs):
        selfc = [selfb[n, pl.ds(c * 16, 16)] for c in range(8)]
        lo = [jnp.zeros((16,), jnp.float32)] * 8
        hi = [jnp.zeros((16,), jnp.float32)] * 8
        for r in range(3):
            sums = _row_sums(gv, r, 8)
            for c in range(8):
                mean = sums[c] * invs[r]
                lo[c] = lo[c] + mean * w2v[r, pl.ds(c * 16, 16)]
                hi[c] = hi[c] + (selfc[c] - mean) * w2v[r, pl.ds(128 + c * 16, 16)]
        for c in range(4):
            outv[n, pl.ds(c * 16, 16)] = selfa[n, pl.ds(c * 16, 16)]
        for c in range(8):
            outv[n, pl.ds(64 + c * 16, 16)] = selfc[c]
            outv[n, pl.ds(192 + c * 16, 16)] = lo[c]
            outv[n, pl.ds(320 + c * 16, 16)] = hi[c]

    inv_a = _dedup_fire(adjv, jnp.int32(0), idx0, g0, sem0, sh_table)

    def body(i2, carry):
        n = i2 * jnp.int32(2)
        inv_b = _dedup_fire(adjv, n + 1, idx1, g1, sem1, sh_table)
        pltpu.make_async_copy(sh_table.at[idx0], g0, sem0).wait()
        consume(n, g0, carry)
        inv_c = _dedup_fire(adjv, n + 2, idx0, g0, sem0, sh_table)
        pltpu.make_async_copy(sh_table.at[idx1], g1, sem1).wait()
        consume(n + 1, g1, inv_b)
        return inv_c

    lax.fori_loop(jnp.int32(0), jnp.int32(C2 // 2), body, inv_a)
    pltpu.make_async_copy(sh_table.at[idx0], g0, sem0).wait()
    pltpu.sync_copy(outv, out_hbm.at[pl.ds(base, C2)])


def _layer2(nodes32, adjcat, table1, table2, w2t):
    f = functools.partial(
        pl.kernel,
        out_type=jax.ShapeDtypeStruct((B, 448), jnp.float32),
        mesh=_mesh(),
        scratch_types=[
            pltpu.VMEM_SHARED((NP, 128), jnp.float32),
            pltpu.VMEM((C2,), jnp.int32),
            pltpu.VMEM((C2 + 2, 48), jnp.int32),
            pltpu.VMEM((C2, 64), jnp.float32),
            pltpu.VMEM((C2, 128), jnp.float32),
            pltpu.VMEM((C2, 448), jnp.float32),
            pltpu.VMEM((3, 256), jnp.float32),
            pltpu.VMEM((48,), jnp.int32),
            pltpu.VMEM((48,), jnp.int32),
            pltpu.VMEM((48, 128), jnp.float32),
            pltpu.VMEM((48, 128), jnp.float32),
            pltpu.SemaphoreType.DMA,
            pltpu.SemaphoreType.DMA,
        ],
        **_SC_PARAMS,
    )(_layer2_body)
    return f(nodes32, adjcat, table1, table2, w2t)


def _final_body(x_ref, w_ref, p_ref, o_ref):
    o_ref[...] = jnp.dot(x_ref[...], w_ref[...],
                         preferred_element_type=jnp.float32) + jnp.log(p_ref[...])


def _final(out2, linear_W, prior):
    return pl.pallas_call(
        _final_body,
        out_shape=jax.ShapeDtypeStruct((B, 2), jnp.float32),
    )(out2, linear_W, prior.reshape(1, 2))


@jax.jit
def kernel(nodes, adj_lists, feat_data, prior, mlp_W, alpha1, alpha2, linear_W):
    nodes32 = nodes.astype(jnp.int32)
    adj32 = adj_lists.astype(jnp.int32)
    adjcat = jnp.transpose(adj32, (1, 0, 2)).reshape(N, 3 * DEG)
    adjcat = jnp.pad(adjcat, ((0, NP - N), (0, 0)), constant_values=ZERO)
    featp = jnp.pad(feat_data, ((0, NP - N), (0, 32 - 25)))
    wp = jnp.pad(mlp_W, ((0, 32 - 25), (0, 0)))
    table1, w1t, w2t = _prep(featp, wp, alpha1.T, alpha2.T)
    table2 = _layer1(adjcat, table1, w1t)
    out2 = _layer2(nodes32, adjcat, table1, table2, w2t)
    y = _final(out2, linear_W, prior)
    return y.astype(jnp.float64)


# speedup vs baseline: 5.1730x; 1.0190x over previous
import functools

import jax
import jax.numpy as jnp
from jax import lax
from jax.experimental import pallas as pl
from jax.experimental.pallas import tpu as pltpu
from jax.experimental.pallas import tpu_sc as plsc

N = 10000
NP = 10240
DEG = 16
ZERO = N
NC, NS = 2, 16
NW = NC * NS
C1 = NP // NW
SEG = NP // NS
B = 256
C2 = B // NW

_SC_PARAMS = dict(
    compiler_params=pltpu.CompilerParams(
        needs_layout_passes=False, use_tc_tiling_on_sc=False))


def _mesh():
    return plsc.VectorSubcoreMesh(
        core_axis_name="c", subcore_axis_name="s", num_cores=NC, num_subcores=NS)


def _wid():
    return lax.axis_index("s") * NC + lax.axis_index("c")


def _prep_body(feat_ref, w_ref, a1_ref, a2_ref, p_ref,
               mlp_ref, w1_ref, w2_ref, lp_ref):
    mlp_ref[...] = jnp.dot(feat_ref[...], w_ref[...],
                           preferred_element_type=jnp.float32)
    for a_ref, o_ref in ((a1_ref, w1_ref), (a2_ref, w2_ref)):
        a = a_ref[...]
        e = jnp.exp(a - jnp.max(a, axis=0, keepdims=True))
        o_ref[...] = e / jnp.sum(e, axis=0, keepdims=True)
    lp_ref[...] = jnp.log(p_ref[...])


def _prep(featp, wp, a1t, a2t, priorp):
    return pl.pallas_call(
        _prep_body,
        out_shape=[
            jax.ShapeDtypeStruct((NP, 64), jnp.float32),
            jax.ShapeDtypeStruct((3, 128), jnp.float32),
            jax.ShapeDtypeStruct((3, 256), jnp.float32),
            jax.ShapeDtypeStruct((1, 16), jnp.float32),
        ],
    )(featp, wp, a1t, a2t, priorp)


def _stage_shared(table_hbm, sh_table):
    seg = lax.axis_index("s") * jnp.int32(SEG)
    pltpu.sync_copy(table_hbm.at[pl.ds(seg, SEG)], sh_table.at[pl.ds(seg, SEG)])
    plsc.subcore_barrier()


def _dedup_fire(adjv, n, idxv, gv, sem, src_table):
    invs = []
    for r in range(3):
        a = adjv[n, pl.ds(r * 16, 16)]
        _, m = plsc.scan_count(a)
        cnt = plsc.all_reduce_population_count(m)
        invs.append(1.0 / cnt.astype(jnp.float32))
        idxv[pl.ds(r * 16, 16)] = jnp.where(m, a, ZERO)
    pltpu.async_copy(src_table.at[idxv], gv, sem)
    return tuple(invs)


def _row_sums(gv, r, ngrp):
    acc = []
    for c in range(ngrp):
        a0 = gv[r * 16, pl.ds(c * 16, 16)]
        for j in range(1, 16):
            a0 = a0 + gv[r * 16 + j, pl.ds(c * 16, 16)]
        acc.append(a0)
    return acc


def _layer1_body(adj_hbm, table_hbm, w1_hbm, out_hbm,
                 sh_table, adjv, selfv, outv, w1v,
                 idx0, idx1, g0, g1, sem0, sem1):
    base = _wid() * C1
    pltpu.async_copy(adj_hbm.at[pl.ds(base, C1)], adjv.at[pl.ds(0, C1)], sem0)
    pltpu.async_copy(table_hbm.at[pl.ds(base, C1)], selfv, sem1)
    pltpu.sync_copy(w1_hbm, w1v)
    _stage_shared(table_hbm, sh_table)
    pltpu.make_async_copy(adj_hbm.at[pl.ds(base, C1)], adjv.at[pl.ds(0, C1)],
                          sem0).wait()
    pltpu.make_async_copy(table_hbm.at[pl.ds(base, C1)], selfv, sem1).wait()
    zvec = jnp.full((16,), ZERO, jnp.int32)
    for rr in range(C1, C1 + 2):
        for c in range(3):
            adjv[rr, pl.ds(c * 16, 16)] = zvec

    wlo = [[w1v[r, pl.ds(c * 16, 16)] for c in range(4)] for r in range(3)]
    whi = [[w1v[r, pl.ds(64 + c * 16, 16)] for c in range(4)] for r in range(3)]

    def consume(n, gv, invs):
        selfc = [selfv[n, pl.ds(c * 16, 16)] for c in range(4)]
        lo = [jnp.zeros((16,), jnp.float32)] * 4
        hi = [jnp.zeros((16,), jnp.float32)] * 4
        for r in range(3):
            sums = _row_sums(gv, r, 4)
            for c in range(4):
                mean = sums[c] * invs[r]
                lo[c] = lo[c] + mean * wlo[r][c]
                hi[c] = hi[c] + (selfc[c] - mean) * whi[r][c]
        for c in range(4):
            outv[n, pl.ds(c * 16, 16)] = lo[c]
            outv[n, pl.ds(64 + c * 16, 16)] = hi[c]

    inv_a = _dedup_fire(adjv, jnp.int32(0), idx0, g0, sem0, sh_table)

    def body(i2, carry):
        n = i2 * jnp.int32(2)
        inv_b = _dedup_fire(adjv, n + 1, idx1, g1, sem1, sh_table)
        pltpu.make_async_copy(sh_table.at[idx0], g0, sem0).wait()
        consume(n, g0, carry)
        inv_c = _dedup_fire(adjv, n + 2, idx0, g0, sem0, sh_table)
        pltpu.make_async_copy(sh_table.at[idx1], g1, sem1).wait()
        consume(n + 1, g1, inv_b)
        return inv_c

    lax.fori_loop(jnp.int32(0), jnp.int32(C1 // 2), body, inv_a)
    pltpu.make_async_copy(sh_table.at[idx0], g0, sem0).wait()
    pltpu.sync_copy(outv, out_hbm.at[pl.ds(base, C1)])


def _layer1(adjcat, table1, w1t):
    f = functools.partial(
        pl.kernel,
        out_type=jax.ShapeDtypeStruct((NP, 128), jnp.float32),
        mesh=_mesh(),
        scratch_types=[
            pltpu.VMEM_SHARED((NP, 64), jnp.float32),
            pltpu.VMEM((C1 + 2, 48), jnp.int32),
            pltpu.VMEM((C1, 64), jnp.float32),
            pltpu.VMEM((C1, 128), jnp.float32),
            pltpu.VMEM((3, 128), jnp.float32),
            pltpu.VMEM((48,), jnp.int32),
            pltpu.VMEM((48,), jnp.int32),
            pltpu.VMEM((48, 64), jnp.float32),
            pltpu.VMEM((48, 64), jnp.float32),
            pltpu.SemaphoreType.DMA,
            pltpu.SemaphoreType.DMA,
        ],
        **_SC_PARAMS,
    )(_layer1_body)
    return f(adjcat, table1, w1t)


def _layer2_body(nodes_hbm, adj_hbm, t1_hbm, t2_hbm, w2_hbm, lw_hbm, lp_hbm,
                 out_hbm,
                 sh_table, nodesv, adjv, selfa, selfb, outv, w2v, lwv, lpv,
                 idx0, idx1, g0, g1, sem0, sem1):
    base = _wid() * C2
    pltpu.sync_copy(nodes_hbm.at[pl.ds(base, C2)], nodesv)
    pltpu.sync_copy(w2_hbm, w2v)
    pltpu.sync_copy(lw_hbm, lwv)
    pltpu.sync_copy(lp_hbm, lpv)
    pltpu.async_copy(adj_hbm.at[nodesv], adjv.at[pl.ds(0, C2)], sem0)
    pltpu.async_copy(t1_hbm.at[nodesv], selfa, sem1)
    pltpu.async_copy(t2_hbm.at[nodesv], selfb, sem1)
    _stage_shared(t2_hbm, sh_table)
    pltpu.make_async_copy(adj_hbm.at[nodesv], adjv.at[pl.ds(0, C2)], sem0).wait()
    pltpu.make_async_copy(t1_hbm.at[nodesv], selfa, sem1).wait()
    pltpu.make_async_copy(t2_hbm.at[nodesv], selfb, sem1).wait()
    zvec = jnp.full((16,), ZERO, jnp.int32)
    for rr in range(C2, C2 + 2):
        for c in range(3):
            adjv[rr, pl.ds(c * 16, 16)] = zvec

    def consume(n, gv, invs):
        selfc = [selfb[n, pl.ds(c * 16, 16)] for c in range(8)]
        lo = [jnp.zeros((16,), jnp.float32)] * 8
        hi = [jnp.zeros((16,), jnp.float32)] * 8
        for r in range(3):
            sums = _row_sums(gv, r, 8)
            for c in range(8):
                mean = sums[c] * invs[r]
                lo[c] = lo[c] + mean * w2v[r, pl.ds(c * 16, 16)]
                hi[c] = hi[c] + (selfc[c] - mean) * w2v[r, pl.ds(128 + c * 16, 16)]
        pieces = ([selfa[n, pl.ds(c * 16, 16)] for c in range(4)]
                  + selfc + lo + hi)
        acc0 = jnp.zeros((16,), jnp.float32)
        acc1 = jnp.zeros((16,), jnp.float32)
        for k in range(28):
            acc0 = acc0 + pieces[k] * lwv[0, pl.ds(k * 16, 16)]
            acc1 = acc1 + pieces[k] * lwv[1, pl.ds(k * 16, 16)]
        s0 = jnp.sum(acc0)
        s1 = jnp.sum(acc1)
        iota = lax.iota(jnp.int32, 16)
        y = jnp.where(iota == 0, s0, jnp.where(iota == 1, s1, 0.0))
        outv[n, :] = y + lpv[0, :]

    inv_a = _dedup_fire(adjv, jnp.int32(0), idx0, g0, sem0, sh_table)

    def body(i2, carry):
        n = i2 * jnp.int32(2)
        inv_b = _dedup_fire(adjv, n + 1, idx1, g1, sem1, sh_table)
        pltpu.make_async_copy(sh_table.at[idx0], g0, sem0).wait()
        consume(n, g0, carry)
        inv_c = _dedup_fire(adjv, n + 2, idx0, g0, sem0, sh_table)
        pltpu.make_async_copy(sh_table.at[idx1], g1, sem1).wait()
        consume(n + 1, g1, inv_b)
        return inv_c

    lax.fori_loop(jnp.int32(0), jnp.int32(C2 // 2), body, inv_a)
    pltpu.make_async_copy(sh_table.at[idx0], g0, sem0).wait()
    pltpu.sync_copy(outv, out_hbm.at[pl.ds(base, C2)])


def _layer2(nodes32, adjcat, table1, table2, w2t, lwt, lp):
    f = functools.partial(
        pl.kernel,
        out_type=jax.ShapeDtypeStruct((B, 16), jnp.float32),
        mesh=_mesh(),
        scratch_types=[
            pltpu.VMEM_SHARED((NP, 128), jnp.float32),
            pltpu.VMEM((C2,), jnp.int32),
            pltpu.VMEM((C2 + 2, 48), jnp.int32),
            pltpu.VMEM((C2, 64), jnp.float32),
            pltpu.VMEM((C2, 128), jnp.float32),
            pltpu.VMEM((C2, 16), jnp.float32),
            pltpu.VMEM((3, 256), jnp.float32),
            pltpu.VMEM((2, 448), jnp.float32),
            pltpu.VMEM((1, 16), jnp.float32),
            pltpu.VMEM((48,), jnp.int32),
            pltpu.VMEM((48,), jnp.int32),
            pltpu.VMEM((48, 128), jnp.float32),
            pltpu.VMEM((48, 128), jnp.float32),
            pltpu.SemaphoreType.DMA,
            pltpu.SemaphoreType.DMA,
        ],
        **_SC_PARAMS,
    )(_layer2_body)
    return f(nodes32, adjcat, table1, table2, w2t, lwt, lp)


@jax.jit
def kernel(nodes, adj_lists, feat_data, prior, mlp_W, alpha1, alpha2, linear_W):
    nodes32 = nodes.astype(jnp.int32)
    adj32 = adj_lists.astype(jnp.int32)
    adjcat = jnp.transpose(adj32, (1, 0, 2)).reshape(N, 3 * DEG)
    adjcat = jnp.pad(adjcat, ((0, NP - N), (0, 0)), constant_values=ZERO)
    featp = jnp.pad(feat_data, ((0, NP - N), (0, 32 - 25)))
    wp = jnp.pad(mlp_W, ((0, 32 - 25), (0, 0)))
    priorp = jnp.pad(prior.reshape(1, 2), ((0, 0), (0, 14)),
                     constant_values=1.0)
    table1, w1t, w2t, lp = _prep(featp, wp, alpha1.T, alpha2.T, priorp)
    table2 = _layer1(adjcat, table1, w1t)
    out16 = _layer2(nodes32, adjcat, table1, table2, w2t,
                    jnp.transpose(linear_W), lp)
    return out16[:, :2].astype(jnp.float64)
